# baseline (speedup 1.0000x reference)
import jax
import jax.numpy as jnp
from jax import lax
from jax.experimental import pallas as pl
from jax.experimental.pallas import tpu as pltpu
from jax.experimental.pallas import tpu_sc as plsc

_VOCAB = 100000
_HASH_SIZE = 4194304
_B = 4096
_L = 50
_NPAIR = _B * (_L - 1)
_NW = 32
_PER_W = _NPAIR // _NW
_VECS = _PER_W // 16


def _lm_body(prev_hbm, cur_hbm, uni_hbm, bo_hbm, bg_hbm, fnd_hbm, out_hbm,
             prev_v, cur_v, h_v, uni_v, bo_v, bg_v, fnd_v, out_v,
             s_p, s_c, s_uni, s_bo, s_bg, s_fnd):
    sid = lax.axis_index("s")
    wid = sid * 2 + lax.axis_index("c")
    base = wid * _PER_W

    cp_p = pltpu.async_copy(prev_hbm.at[pl.ds(base, _PER_W)], prev_v, s_p)
    cp_c = pltpu.async_copy(cur_hbm.at[pl.ds(base, _PER_W)], cur_v, s_c)
    with jax.named_scope("wait_ids"):
        cp_p.wait()
        cp_c.wait()

    cp_bo = pltpu.async_copy(bo_hbm.at[prev_v], bo_v, s_bo)
    cp_uni = pltpu.async_copy(uni_hbm.at[cur_v], uni_v, s_uni)

    def hash_chunk(i, _):
        sl = pl.ds(i * 16, 16)
        h_v[sl] = (prev_v[sl] * 1000003 + cur_v[sl]) & (_HASH_SIZE - 1)
        return 0

    with jax.named_scope("hash"):
        lax.fori_loop(0, _VECS, hash_chunk, 0)

    cp_bg = pltpu.async_copy(bg_hbm.at[h_v], bg_v, s_bg)
    cp_fnd = pltpu.async_copy(fnd_hbm.at[h_v], fnd_v, s_fnd)

    with jax.named_scope("wait_gathers"):
        cp_uni.wait()
        cp_bo.wait()
        cp_bg.wait()
        cp_fnd.wait()

    def blend_chunk(i, _):
        sl = pl.ds(i * 16, 16)
        f = fnd_v[sl]
        out_v[sl] = f * bg_v[sl] + (1.0 - f) * (bo_v[sl] + uni_v[sl])
        return 0

    with jax.named_scope("blend"):
        lax.fori_loop(0, _VECS, blend_chunk, 0)

    with jax.named_scope("out"):
        pltpu.sync_copy(out_v, out_hbm.at[pl.ds(base, _PER_W)])


@jax.jit
def _lm(prev_flat, cur_flat, uni, bo, bg, fnd):
    run = pl.kernel(
        _lm_body,
        out_type=jax.ShapeDtypeStruct((_NPAIR,), jnp.float32),
        mesh=plsc.VectorSubcoreMesh(core_axis_name="c", subcore_axis_name="s"),
        scratch_types=[
            pltpu.VMEM((_PER_W,), jnp.int32),
            pltpu.VMEM((_PER_W,), jnp.int32),
            pltpu.VMEM((_PER_W,), jnp.int32),
            pltpu.VMEM((_PER_W,), jnp.float32),
            pltpu.VMEM((_PER_W,), jnp.float32),
            pltpu.VMEM((_PER_W,), jnp.float32),
            pltpu.VMEM((_PER_W,), jnp.float32),
            pltpu.VMEM((_PER_W,), jnp.float32),
            pltpu.SemaphoreType.DMA,
            pltpu.SemaphoreType.DMA,
            pltpu.SemaphoreType.DMA,
            pltpu.SemaphoreType.DMA,
            pltpu.SemaphoreType.DMA,
            pltpu.SemaphoreType.DMA,
        ],
    )
    return run(prev_flat, cur_flat, uni, bo, bg, fnd)


def kernel(x, unigram_logp, unigram_backoff, bigram_logp, bigram_found):
    xi = x.astype(jnp.int32)
    prev_flat = xi[:, :-1].reshape(-1)
    cur_flat = xi[:, 1:].reshape(-1)
    out = _lm(prev_flat, cur_flat, unigram_logp, unigram_backoff,
              bigram_logp, bigram_found)
    return out.reshape(_B, _L - 1)


# speedup vs baseline: 1.0313x; 1.0313x over previous
import jax
import jax.numpy as jnp
from jax import lax
from jax.experimental import pallas as pl
from jax.experimental.pallas import tpu as pltpu
from jax.experimental.pallas import tpu_sc as plsc

_VOCAB = 100000
_HASH_SIZE = 4194304
_B = 4096
_L = 50
_NPAIR = _B * (_L - 1)
_NW = 32
_ROWS_W = _B // _NW
_PER_W = _ROWS_W * (_L - 1)
_CHUNKS = (0, 16, 32, 33)


def _lm_body(x_hbm, uni_hbm, bo_hbm, bg_hbm, fnd_hbm, out_hbm,
             x_v, prev_v, cur_v, h_v, uni_v, bo_v, bg_v, fnd_v, out_v,
             s_x, s_uni, s_bo, s_bg, s_fnd):
    sid = lax.axis_index("s")
    wid = sid * 2 + lax.axis_index("c")
    row0 = wid * _ROWS_W

    cp_x = pltpu.async_copy(x_hbm.at[pl.ds(row0, _ROWS_W)], x_v, s_x)
    with jax.named_scope("wait_x"):
        cp_x.wait()

    def hash_row(r, _):
        p0 = r * (_L - 1)
        for o in _CHUNKS:
            pv = x_v[r, pl.ds(o, 16)]
            cv = x_v[r, pl.ds(o + 1, 16)]
            sl = pl.ds(p0 + o, 16)
            prev_v[sl] = pv
            cur_v[sl] = cv
            h_v[sl] = (pv * 1000003 + cv) & (_HASH_SIZE - 1)
        return 0

    with jax.named_scope("hash"):
        lax.fori_loop(0, _ROWS_W, hash_row, 0)

    cp_bo = pltpu.async_copy(bo_hbm.at[prev_v], bo_v, s_bo)
    cp_uni = pltpu.async_copy(uni_hbm.at[cur_v], uni_v, s_uni)
    cp_bg = pltpu.async_copy(bg_hbm.at[h_v], bg_v, s_bg)
    cp_fnd = pltpu.async_copy(fnd_hbm.at[h_v], fnd_v, s_fnd)

    with jax.named_scope("wait_gathers"):
        cp_uni.wait()
        cp_bo.wait()
        cp_bg.wait()
        cp_fnd.wait()

    def blend_row(r, _):
        p0 = r * (_L - 1)
        for o in _CHUNKS:
            sl = pl.ds(p0 + o, 16)
            f = fnd_v[sl]
            out_v[r, pl.ds(o, 16)] = f * bg_v[sl] + (1.0 - f) * (bo_v[sl] + uni_v[sl])
        return 0

    with jax.named_scope("blend"):
        lax.fori_loop(0, _ROWS_W, blend_row, 0)

    with jax.named_scope("out"):
        pltpu.sync_copy(out_v, out_hbm.at[pl.ds(row0, _ROWS_W)])


@jax.jit
def _lm(x, uni, bo, bg, fnd):
    run = pl.kernel(
        _lm_body,
        out_type=jax.ShapeDtypeStruct((_B, _L - 1), jnp.float32),
        mesh=plsc.VectorSubcoreMesh(core_axis_name="c", subcore_axis_name="s"),
        scratch_types=[
            pltpu.VMEM((_ROWS_W, _L), jnp.int32),
            pltpu.VMEM((_PER_W,), jnp.int32),
            pltpu.VMEM((_PER_W,), jnp.int32),
            pltpu.VMEM((_PER_W,), jnp.int32),
            pltpu.VMEM((_PER_W,), jnp.float32),
            pltpu.VMEM((_PER_W,), jnp.float32),
            pltpu.VMEM((_PER_W,), jnp.float32),
            pltpu.VMEM((_PER_W,), jnp.float32),
            pltpu.VMEM((_ROWS_W, _L - 1), jnp.float32),
            pltpu.SemaphoreType.DMA,
            pltpu.SemaphoreType.DMA,
            pltpu.SemaphoreType.DMA,
            pltpu.SemaphoreType.DMA,
            pltpu.SemaphoreType.DMA,
        ],
    )
    return run(x, uni, bo, bg, fnd)


def kernel(x, unigram_logp, unigram_backoff, bigram_logp, bigram_found):
    return _lm(x.astype(jnp.int32), unigram_logp, unigram_backoff,
               bigram_logp, bigram_found)


# speedup vs baseline: 1.0887x; 1.0557x over previous
import jax
import jax.numpy as jnp
from jax import lax
from jax.experimental import pallas as pl
from jax.experimental.pallas import tpu as pltpu
from jax.experimental.pallas import tpu_sc as plsc

_VOCAB = 100000
_HASH_SIZE = 4194304
_B = 4096
_L = 50
_NPAIR = _B * (_L - 1)
_NW = 32
_ROWS_W = _B // _NW
_ROWS_H = _ROWS_W // 2
_PER_H = _ROWS_H * (_L - 1)
_CHUNKS = (0, 16, 32, 33)


def _lm_body(x_hbm, uni_hbm, bo_hbm, bg_hbm, fnd_hbm, out_hbm,
             x_v, out_v,
             prev_a, cur_a, h_a, uni_a, bo_a, bg_a, fnd_a,
             prev_b, cur_b, h_b, uni_b, bo_b, bg_b, fnd_b,
             s_x, s_ua, s_oa, s_ga, s_fa, s_ub, s_ob, s_gb, s_fb):
    sid = lax.axis_index("s")
    wid = sid * 2 + lax.axis_index("c")
    row0 = wid * _ROWS_W

    cp_x = pltpu.async_copy(x_hbm.at[pl.ds(row0, _ROWS_W)], x_v, s_x)
    with jax.named_scope("wait_x"):
        cp_x.wait()

    def unpack_half(r0, prev_r, cur_r, h_r):
        def body(r, _):
            for o in _CHUNKS:
                pv = x_v[r0 + r, pl.ds(o, 16)]
                cv = x_v[r0 + r, pl.ds(o + 1, 16)]
                sl = pl.ds(r * (_L - 1) + o, 16)
                prev_r[sl] = pv
                cur_r[sl] = cv
                h_r[sl] = (pv * 1000003 + cv) & (_HASH_SIZE - 1)
            return 0
        lax.fori_loop(0, _ROWS_H, body, 0)

    def blend_half(r0, uni_r, bo_r, bg_r, fnd_r):
        def body(r, _):
            for o in _CHUNKS:
                sl = pl.ds(r * (_L - 1) + o, 16)
                f = fnd_r[sl]
                out_v[r0 + r, pl.ds(o, 16)] = (
                    f * bg_r[sl] + (1.0 - f) * (bo_r[sl] + uni_r[sl]))
            return 0
        lax.fori_loop(0, _ROWS_H, body, 0)

    with jax.named_scope("unpack_a"):
        unpack_half(0, prev_a, cur_a, h_a)
    cp_ua = pltpu.async_copy(uni_hbm.at[cur_a], uni_a, s_ua)
    cp_oa = pltpu.async_copy(bo_hbm.at[prev_a], bo_a, s_oa)
    cp_ga = pltpu.async_copy(bg_hbm.at[h_a], bg_a, s_ga)
    cp_fa = pltpu.async_copy(fnd_hbm.at[h_a], fnd_a, s_fa)

    with jax.named_scope("unpack_b"):
        unpack_half(_ROWS_H, prev_b, cur_b, h_b)
    cp_ub = pltpu.async_copy(uni_hbm.at[cur_b], uni_b, s_ub)
    cp_ob = pltpu.async_copy(bo_hbm.at[prev_b], bo_b, s_ob)
    cp_gb = pltpu.async_copy(bg_hbm.at[h_b], bg_b, s_gb)
    cp_fb = pltpu.async_copy(fnd_hbm.at[h_b], fnd_b, s_fb)

    with jax.named_scope("wait_a"):
        cp_ua.wait()
        cp_oa.wait()
        cp_ga.wait()
        cp_fa.wait()
    with jax.named_scope("blend_a"):
        blend_half(0, uni_a, bo_a, bg_a, fnd_a)

    with jax.named_scope("wait_b"):
        cp_ub.wait()
        cp_ob.wait()
        cp_gb.wait()
        cp_fb.wait()
    with jax.named_scope("blend_b"):
        blend_half(_ROWS_H, uni_b, bo_b, bg_b, fnd_b)

    with jax.named_scope("out"):
        pltpu.sync_copy(out_v, out_hbm.at[pl.ds(row0, _ROWS_W)])


@jax.jit
def _lm(x, uni, bo, bg, fnd):
    half_bufs = [
        pltpu.VMEM((_PER_H,), jnp.int32),
        pltpu.VMEM((_PER_H,), jnp.int32),
        pltpu.VMEM((_PER_H,), jnp.int32),
        pltpu.VMEM((_PER_H,), jnp.float32),
        pltpu.VMEM((_PER_H,), jnp.float32),
        pltpu.VMEM((_PER_H,), jnp.float32),
        pltpu.VMEM((_PER_H,), jnp.float32),
    ]
    run = pl.kernel(
        _lm_body,
        out_type=jax.ShapeDtypeStruct((_B, _L - 1), jnp.float32),
        mesh=plsc.VectorSubcoreMesh(core_axis_name="c", subcore_axis_name="s"),
        scratch_types=(
            [pltpu.VMEM((_ROWS_W, _L), jnp.int32),
             pltpu.VMEM((_ROWS_W, _L - 1), jnp.float32)]
            + half_bufs + half_bufs
            + [pltpu.SemaphoreType.DMA] * 9
        ),
    )
    return run(x, uni, bo, bg, fnd)


def kernel(x, unigram_logp, unigram_backoff, bigram_logp, bigram_found):
    return _lm(x.astype(jnp.int32), unigram_logp, unigram_backoff,
               bigram_logp, bigram_found)


# speedup vs baseline: 1.4523x; 1.3339x over previous
import jax
import jax.numpy as jnp
from jax import lax
from jax.experimental import pallas as pl
from jax.experimental.pallas import tpu as pltpu
from jax.experimental.pallas import tpu_sc as plsc

_VOCAB = 100000
_HASH_SIZE = 4194304
_B = 4096
_L = 50
_NPAIR = _B * (_L - 1)
_NW = 32
_ROWS_W = _B // _NW
_ROWS_H = _ROWS_W // 2
_PER_H = _ROWS_H * (_L - 1)
_CHUNKS = (0, 16, 32, 33)
_STAGE = 6256


def _lm_body(x_hbm, uni_hbm, bo_hbm, bg_hbm, fnd_hbm, out_hbm,
             x_v, out_v, bnc_u, bnc_b, uni_sh, bo_sh,
             prev_a, cur_a, h_a, uni_a, bo_a, bg_a, fnd_a,
             prev_b, cur_b, h_b, uni_b, bo_b, bg_b, fnd_b,
             s_x, s_su, s_sb,
             s_ua, s_oa, s_ga, s_fa, s_ub, s_ob, s_gb, s_fb):
    sid = lax.axis_index("s")
    wid = sid * 2 + lax.axis_index("c")
    row0 = wid * _ROWS_W

    sbase = jnp.minimum(sid * _STAGE, _VOCAB - _STAGE)
    cp_su = pltpu.async_copy(uni_hbm.at[pl.ds(sbase, _STAGE)], bnc_u, s_su)
    cp_sb = pltpu.async_copy(bo_hbm.at[pl.ds(sbase, _STAGE)], bnc_b, s_sb)
    cp_x = pltpu.async_copy(x_hbm.at[pl.ds(row0, _ROWS_W)], x_v, s_x)

    with jax.named_scope("publish"):
        cp_su.wait()
        pltpu.sync_copy(bnc_u, uni_sh.at[pl.ds(sbase, _STAGE)])
        cp_sb.wait()
        pltpu.sync_copy(bnc_b, bo_sh.at[pl.ds(sbase, _STAGE)])

    with jax.named_scope("wait_x"):
        cp_x.wait()

    def unpack_half(r0, prev_r, cur_r, h_r):
        def body(r, _):
            for o in _CHUNKS:
                pv = x_v[r0 + r, pl.ds(o, 16)]
                cv = x_v[r0 + r, pl.ds(o + 1, 16)]
                sl = pl.ds(r * (_L - 1) + o, 16)
                prev_r[sl] = pv
                cur_r[sl] = cv
                h_r[sl] = (pv * 1000003 + cv) & (_HASH_SIZE - 1)
            return 0
        lax.fori_loop(0, _ROWS_H, body, 0)

    def blend_half(r0, uni_r, bo_r, bg_r, fnd_r):
        def body(r, _):
            for o in _CHUNKS:
                sl = pl.ds(r * (_L - 1) + o, 16)
                f = fnd_r[sl]
                out_v[r0 + r, pl.ds(o, 16)] = (
                    f * bg_r[sl] + (1.0 - f) * (bo_r[sl] + uni_r[sl]))
            return 0
        lax.fori_loop(0, _ROWS_H, body, 0)

    with jax.named_scope("unpack_a"):
        unpack_half(0, prev_a, cur_a, h_a)
    cp_ga = pltpu.async_copy(bg_hbm.at[h_a], bg_a, s_ga)
    cp_fa = pltpu.async_copy(fnd_hbm.at[h_a], fnd_a, s_fa)

    with jax.named_scope("unpack_b"):
        unpack_half(_ROWS_H, prev_b, cur_b, h_b)
    cp_gb = pltpu.async_copy(bg_hbm.at[h_b], bg_b, s_gb)
    cp_fb = pltpu.async_copy(fnd_hbm.at[h_b], fnd_b, s_fb)

    plsc.subcore_barrier()
    cp_ua = pltpu.async_copy(uni_sh.at[cur_a], uni_a, s_ua)
    cp_oa = pltpu.async_copy(bo_sh.at[prev_a], bo_a, s_oa)
    cp_ub = pltpu.async_copy(uni_sh.at[cur_b], uni_b, s_ub)
    cp_ob = pltpu.async_copy(bo_sh.at[prev_b], bo_b, s_ob)

    with jax.named_scope("wait_a"):
        cp_ua.wait()
        cp_oa.wait()
        cp_ga.wait()
        cp_fa.wait()
    with jax.named_scope("blend_a"):
        blend_half(0, uni_a, bo_a, bg_a, fnd_a)

    with jax.named_scope("wait_b"):
        cp_ub.wait()
        cp_ob.wait()
        cp_gb.wait()
        cp_fb.wait()
    with jax.named_scope("blend_b"):
        blend_half(_ROWS_H, uni_b, bo_b, bg_b, fnd_b)

    with jax.named_scope("out"):
        pltpu.sync_copy(out_v, out_hbm.at[pl.ds(row0, _ROWS_W)])


@jax.jit
def _lm(x, uni, bo, bg, fnd):
    half_bufs = [
        pltpu.VMEM((_PER_H,), jnp.int32),
        pltpu.VMEM((_PER_H,), jnp.int32),
        pltpu.VMEM((_PER_H,), jnp.int32),
        pltpu.VMEM((_PER_H,), jnp.float32),
        pltpu.VMEM((_PER_H,), jnp.float32),
        pltpu.VMEM((_PER_H,), jnp.float32),
        pltpu.VMEM((_PER_H,), jnp.float32),
    ]
    run = pl.kernel(
        _lm_body,
        out_type=jax.ShapeDtypeStruct((_B, _L - 1), jnp.float32),
        mesh=plsc.VectorSubcoreMesh(core_axis_name="c", subcore_axis_name="s"),
        scratch_types=(
            [pltpu.VMEM((_ROWS_W, _L), jnp.int32),
             pltpu.VMEM((_ROWS_W, _L - 1), jnp.float32),
             pltpu.VMEM((_STAGE,), jnp.float32),
             pltpu.VMEM((_STAGE,), jnp.float32),
             pltpu.VMEM_SHARED((_VOCAB,), jnp.float32),
             pltpu.VMEM_SHARED((_VOCAB,), jnp.float32)]
            + half_bufs + half_bufs
            + [pltpu.SemaphoreType.DMA] * 11
        ),
    )
    return run(x, uni, bo, bg, fnd)


def kernel(x, unigram_logp, unigram_backoff, bigram_logp, bigram_found):
    return _lm(x.astype(jnp.int32), unigram_logp, unigram_backoff,
               bigram_logp, bigram_found)


# speedup vs baseline: 1.5065x; 1.0373x over previous
import jax
import jax.numpy as jnp
from jax import lax
from jax.experimental import pallas as pl
from jax.experimental.pallas import tpu as pltpu
from jax.experimental.pallas import tpu_sc as plsc

_VOCAB = 100000
_HASH_SIZE = 4194304
_B = 4096
_L = 50
_NPAIR = _B * (_L - 1)
_NW = 32
_ROWS_W = _B // _NW
_NQ = 4
_ROWS_Q = _ROWS_W // _NQ
_PER_Q = _ROWS_Q * (_L - 1)
_CHUNKS = (0, 16, 32, 33)
_STAGE = 6256


def _lm_body(x_hbm, uni_hbm, bo_hbm, bg_hbm, fnd_hbm, out_hbm,
             x_v, out_v, bnc_u, bnc_b, uni_sh, bo_sh, *rest):
    bufs = rest[:7 * _NQ]
    sems = rest[7 * _NQ:]
    s_x, s_su, s_sb = sems[0], sems[1], sems[2]
    qsems = sems[3:]

    sid = lax.axis_index("s")
    wid = sid * 2 + lax.axis_index("c")
    row0 = wid * _ROWS_W

    sbase = jnp.minimum(sid * _STAGE, _VOCAB - _STAGE)
    cp_su = pltpu.async_copy(uni_hbm.at[pl.ds(sbase, _STAGE)], bnc_u, s_su)
    cp_sb = pltpu.async_copy(bo_hbm.at[pl.ds(sbase, _STAGE)], bnc_b, s_sb)
    cp_x = pltpu.async_copy(x_hbm.at[pl.ds(row0, _ROWS_W)], x_v, s_x)

    with jax.named_scope("publish"):
        cp_su.wait()
        pltpu.sync_copy(bnc_u, uni_sh.at[pl.ds(sbase, _STAGE)])
        cp_sb.wait()
        pltpu.sync_copy(bnc_b, bo_sh.at[pl.ds(sbase, _STAGE)])

    with jax.named_scope("wait_x"):
        cp_x.wait()

    def unpack_quarter(r0, prev_r, cur_r, h_r):
        def body(r, _):
            for o in _CHUNKS:
                pv = x_v[r0 + r, pl.ds(o, 16)]
                cv = x_v[r0 + r, pl.ds(o + 1, 16)]
                sl = pl.ds(r * (_L - 1) + o, 16)
                prev_r[sl] = pv
                cur_r[sl] = cv
                h_r[sl] = (pv * 1000003 + cv) & (_HASH_SIZE - 1)
            return 0
        lax.fori_loop(0, _ROWS_Q, body, 0)

    def blend_quarter(r0, uni_r, bo_r, bg_r, fnd_r):
        def body(r, _):
            for o in _CHUNKS:
                sl = pl.ds(r * (_L - 1) + o, 16)
                f = fnd_r[sl]
                out_v[r0 + r, pl.ds(o, 16)] = (
                    f * bg_r[sl] + (1.0 - f) * (bo_r[sl] + uni_r[sl]))
            return 0
        lax.fori_loop(0, _ROWS_Q, body, 0)

    copies = []
    for q in range(_NQ):
        prev_r, cur_r, h_r, uni_r, bo_r, bg_r, fnd_r = bufs[7 * q:7 * q + 7]
        s_u, s_o, s_g, s_f = qsems[4 * q:4 * q + 4]
        with jax.named_scope("unpack"):
            unpack_quarter(q * _ROWS_Q, prev_r, cur_r, h_r)
        cp_g = pltpu.async_copy(bg_hbm.at[h_r], bg_r, s_g)
        cp_f = pltpu.async_copy(fnd_hbm.at[h_r], fnd_r, s_f)
        if q == 0:
            plsc.subcore_barrier()
        cp_u = pltpu.async_copy(uni_sh.at[cur_r], uni_r, s_u)
        cp_o = pltpu.async_copy(bo_sh.at[prev_r], bo_r, s_o)
        copies.append((cp_u, cp_o, cp_g, cp_f))

    for q in range(_NQ):
        _, _, _, uni_r, bo_r, bg_r, fnd_r = bufs[7 * q:7 * q + 7]
        cp_u, cp_o, cp_g, cp_f = copies[q]
        with jax.named_scope("wait_q"):
            cp_u.wait()
            cp_o.wait()
            cp_g.wait()
            cp_f.wait()
        with jax.named_scope("blend"):
            blend_quarter(q * _ROWS_Q, uni_r, bo_r, bg_r, fnd_r)

    with jax.named_scope("out"):
        pltpu.sync_copy(out_v, out_hbm.at[pl.ds(row0, _ROWS_W)])


@jax.jit
def _lm(x, uni, bo, bg, fnd):
    quarter_bufs = [
        pltpu.VMEM((_PER_Q,), jnp.int32),
        pltpu.VMEM((_PER_Q,), jnp.int32),
        pltpu.VMEM((_PER_Q,), jnp.int32),
        pltpu.VMEM((_PER_Q,), jnp.float32),
        pltpu.VMEM((_PER_Q,), jnp.float32),
        pltpu.VMEM((_PER_Q,), jnp.float32),
        pltpu.VMEM((_PER_Q,), jnp.float32),
    ]
    run = pl.kernel(
        _lm_body,
        out_type=jax.ShapeDtypeStruct((_B, _L - 1), jnp.float32),
        mesh=plsc.VectorSubcoreMesh(core_axis_name="c", subcore_axis_name="s"),
        scratch_types=(
            [pltpu.VMEM((_ROWS_W, _L), jnp.int32),
             pltpu.VMEM((_ROWS_W, _L - 1), jnp.float32),
             pltpu.VMEM((_STAGE,), jnp.float32),
             pltpu.VMEM((_STAGE,), jnp.float32),
             pltpu.VMEM_SHARED((_VOCAB,), jnp.float32),
             pltpu.VMEM_SHARED((_VOCAB,), jnp.float32)]
            + quarter_bufs * _NQ
            + [pltpu.SemaphoreType.DMA] * (3 + 4 * _NQ)
        ),
    )
    return run(x, uni, bo, bg, fnd)


def kernel(x, unigram_logp, unigram_backoff, bigram_logp, bigram_found):
    return _lm(x.astype(jnp.int32), unigram_logp, unigram_backoff,
               bigram_logp, bigram_found)


# speedup vs baseline: 1.5257x; 1.0128x over previous
import jax
import jax.numpy as jnp
from jax import lax
from jax.experimental import pallas as pl
from jax.experimental.pallas import tpu as pltpu
from jax.experimental.pallas import tpu_sc as plsc

_VOCAB = 100000
_HASH_SIZE = 4194304
_B = 4096
_L = 50
_NPAIR = _B * (_L - 1)
_NW = 32
_ROWS_W = _B // _NW
_NQ = 4
_ROWS_Q = _ROWS_W // _NQ
_PER_Q = _ROWS_Q * (_L - 1)
_CHUNKS = (0, 16, 32, 33)
_STAGE = 6256


def _lm_body(x_hbm, uni_hbm, bo_hbm, bg_hbm, fnd_hbm, out_hbm,
             x_v, out_v, bnc_u, bnc_b, uni_sh, bo_sh, *rest):
    bufs = rest[:7 * _NQ]
    sems = rest[7 * _NQ:]
    s_x, s_su, s_sb = sems[0], sems[1], sems[2]
    qsems = sems[3:]

    sid = lax.axis_index("s")
    wid = sid * 2 + lax.axis_index("c")
    row0 = wid * _ROWS_W

    sbase = jnp.minimum(sid * _STAGE, _VOCAB - _STAGE)
    cp_su = pltpu.async_copy(uni_hbm.at[pl.ds(sbase, _STAGE)], bnc_u, s_su)
    cp_sb = pltpu.async_copy(bo_hbm.at[pl.ds(sbase, _STAGE)], bnc_b, s_sb)
    cp_x = pltpu.async_copy(x_hbm.at[pl.ds(row0, _ROWS_W)], x_v, s_x)

    with jax.named_scope("wait_x"):
        cp_x.wait()

    def unpack_quarter(r0, prev_r, cur_r, h_r):
        def body(r, _):
            for o in _CHUNKS:
                pv = x_v[r0 + r, pl.ds(o, 16)]
                cv = x_v[r0 + r, pl.ds(o + 1, 16)]
                sl = pl.ds(r * (_L - 1) + o, 16)
                prev_r[sl] = pv
                cur_r[sl] = cv
                h_r[sl] = (pv * 1000003 + cv) & (_HASH_SIZE - 1)
            return 0
        lax.fori_loop(0, _ROWS_Q, body, 0)

    def blend_quarter(r0, uni_r, bo_r, bg_r, fnd_r):
        def body(r, _):
            for o in _CHUNKS:
                sl = pl.ds(r * (_L - 1) + o, 16)
                f = fnd_r[sl]
                out_v[r0 + r, pl.ds(o, 16)] = (
                    f * bg_r[sl] + (1.0 - f) * (bo_r[sl] + uni_r[sl]))
            return 0
        lax.fori_loop(0, _ROWS_Q, body, 0)

    copies = []
    for q in range(_NQ):
        prev_r, cur_r, h_r, uni_r, bo_r, bg_r, fnd_r = bufs[7 * q:7 * q + 7]
        s_u, s_o, s_g, s_f = qsems[4 * q:4 * q + 4]
        with jax.named_scope("unpack"):
            unpack_quarter(q * _ROWS_Q, prev_r, cur_r, h_r)
        cp_g = pltpu.async_copy(bg_hbm.at[h_r], bg_r, s_g)
        cp_f = pltpu.async_copy(fnd_hbm.at[h_r], fnd_r, s_f)
        if q == 0:
            with jax.named_scope("publish"):
                cp_su.wait()
                pltpu.sync_copy(bnc_u, uni_sh.at[pl.ds(sbase, _STAGE)])
                cp_sb.wait()
                pltpu.sync_copy(bnc_b, bo_sh.at[pl.ds(sbase, _STAGE)])
            plsc.subcore_barrier()
        cp_u = pltpu.async_copy(uni_sh.at[cur_r], uni_r, s_u)
        cp_o = pltpu.async_copy(bo_sh.at[prev_r], bo_r, s_o)
        copies.append((cp_u, cp_o, cp_g, cp_f))

    for q in range(_NQ):
        _, _, _, uni_r, bo_r, bg_r, fnd_r = bufs[7 * q:7 * q + 7]
        cp_u, cp_o, cp_g, cp_f = copies[q]
        with jax.named_scope("wait_q"):
            cp_u.wait()
            cp_o.wait()
            cp_g.wait()
            cp_f.wait()
        with jax.named_scope("blend"):
            blend_quarter(q * _ROWS_Q, uni_r, bo_r, bg_r, fnd_r)

    with jax.named_scope("out"):
        pltpu.sync_copy(out_v, out_hbm.at[pl.ds(row0, _ROWS_W)])


@jax.jit
def _lm(x, uni, bo, bg, fnd):
    quarter_bufs = [
        pltpu.VMEM((_PER_Q,), jnp.int32),
        pltpu.VMEM((_PER_Q,), jnp.int32),
        pltpu.VMEM((_PER_Q,), jnp.int32),
        pltpu.VMEM((_PER_Q,), jnp.float32),
        pltpu.VMEM((_PER_Q,), jnp.float32),
        pltpu.VMEM((_PER_Q,), jnp.float32),
        pltpu.VMEM((_PER_Q,), jnp.float32),
    ]
    run = pl.kernel(
        _lm_body,
        out_type=jax.ShapeDtypeStruct((_B, _L - 1), jnp.float32),
        mesh=plsc.VectorSubcoreMesh(core_axis_name="c", subcore_axis_name="s"),
        scratch_types=(
            [pltpu.VMEM((_ROWS_W, _L), jnp.int32),
             pltpu.VMEM((_ROWS_W, _L - 1), jnp.float32),
             pltpu.VMEM((_STAGE,), jnp.float32),
             pltpu.VMEM((_STAGE,), jnp.float32),
             pltpu.VMEM_SHARED((_VOCAB,), jnp.float32),
             pltpu.VMEM_SHARED((_VOCAB,), jnp.float32)]
            + quarter_bufs * _NQ
            + [pltpu.SemaphoreType.DMA] * (3 + 4 * _NQ)
        ),
    )
    return run(x, uni, bo, bg, fnd)


def kernel(x, unigram_logp, unigram_backoff, bigram_logp, bigram_found):
    return _lm(x.astype(jnp.int32), unigram_logp, unigram_backoff,
               bigram_logp, bigram_found)
